# Initial kernel scaffold; baseline (speedup 1.0000x reference)
import jax
import jax.numpy as jnp
from jax.experimental import pallas as pl


def kernel(x, idxs, weights):
    raise NotImplementedError("write your pallas kernel here")


# speedup vs baseline: 7.6654x; 7.6654x over previous
import functools

import jax
import jax.numpy as jnp
from jax import lax
from jax.experimental import pallas as pl
from jax.experimental.pallas import tpu as pltpu
from jax.experimental.pallas import tpu_sc as plsc

B = 128
NI = 8192
NN = 16384
FI = 16
NC = 2
NS = 16
NW = NC * NS
NPT = NN // NW
CH = 8
NCH = NPT // CH
ROWS = CH * FI
LANES = 16
NV = B // LANES


def _prep_body(xt_ref, w_ref, e_ref, sw_ref):
    e_ref[...] = jnp.exp(xt_ref[...])
    w = w_ref[...]
    m = jnp.max(w, axis=-1, keepdims=True)
    ew = jnp.exp(w - m)
    sw_ref[...] = ew / jnp.sum(ew, axis=-1, keepdims=True)


def _prep(xt, weights):
    return pl.pallas_call(
        _prep_body,
        out_shape=(
            jax.ShapeDtypeStruct((NI, B), jnp.float32),
            jax.ShapeDtypeStruct((NN, FI), jnp.float32),
        ),
    )(xt, weights)


def _log_body(s_ref, o_ref):
    o_ref[...] = jnp.log(s_ref[...])


def _log_kernel(s):
    return pl.pallas_call(
        _log_body,
        grid=(8,),
        in_specs=[pl.BlockSpec((NN // 8, B), lambda i: (i, 0))],
        out_specs=pl.BlockSpec((NN // 8, B), lambda i: (i, 0)),
        out_shape=jax.ShapeDtypeStruct((NN, B), jnp.float32),
    )(s)


_sc_mesh = plsc.VectorSubcoreMesh(
    core_axis_name="c", subcore_axis_name="s", num_cores=NC, num_subcores=NS
)


@functools.partial(
    pl.kernel,
    out_type=jax.ShapeDtypeStruct((NN, B), jnp.float32),
    mesh=_sc_mesh,
    scratch_types=[
        pltpu.VMEM((NCH, ROWS), jnp.int32),
        pltpu.VMEM((NPT, FI), jnp.float32),
        pltpu.VMEM((ROWS, B), jnp.float32),
        pltpu.VMEM((ROWS, B), jnp.float32),
        pltpu.VMEM((CH, B), jnp.float32),
        pltpu.SemaphoreType.DMA,
        pltpu.SemaphoreType.DMA,
    ],
)
def _sc_gather_reduce(e_hbm, idx_hbm, w_hbm, out_hbm,
                      idx_v, w_v, rows0, rows1, acc_v, sem0, sem1):
    wid = lax.axis_index("s") * NC + lax.axis_index("c")
    pltpu.sync_copy(idx_hbm.at[wid], idx_v)
    pltpu.sync_copy(w_hbm.at[wid], w_v)

    def compute_chunk(j, rows_ref):
        def node_body(n, carry):
            w_vec = w_v[j * CH + n, :]
            acc = [jnp.zeros((LANES,), jnp.float32) for _ in range(NV)]
            for k in range(FI):
                wb = jnp.take(w_vec, jnp.full((LANES,), k, jnp.int32), axis=0)
                r = n * FI + k
                for v in range(NV):
                    acc[v] = acc[v] + wb * rows_ref[r, pl.ds(v * LANES, LANES)]
            for v in range(NV):
                acc_v[n, pl.ds(v * LANES, LANES)] = acc[v]
            return carry
        lax.fori_loop(0, CH, node_body, 0)
        pltpu.sync_copy(acc_v, out_hbm.at[pl.ds(wid * NPT + j * CH, CH)])

    pltpu.async_copy(e_hbm.at[idx_v.at[0]], rows0, sem0)

    def pair_body(i, carry):
        j = 2 * i
        pltpu.async_copy(e_hbm.at[idx_v.at[j + 1]], rows1, sem1)
        pltpu.make_async_copy(e_hbm.at[idx_v.at[j]], rows0, sem0).wait()
        compute_chunk(j, rows0)

        @pl.when(i < NCH // 2 - 1)
        def _():
            pltpu.async_copy(e_hbm.at[idx_v.at[j + 2]], rows0, sem0)

        pltpu.make_async_copy(e_hbm.at[idx_v.at[j + 1]], rows1, sem1).wait()
        compute_chunk(j + 1, rows1)
        return carry

    lax.fori_loop(0, NCH // 2, pair_body, 0)


def kernel(x, idxs, weights):
    xt = jnp.transpose(x)
    et, sw = _prep(xt, weights)
    idx32 = idxs.astype(jnp.int32).reshape(NW, NCH, ROWS)
    swr = sw.reshape(NW, NPT, FI)
    st = _sc_gather_reduce(et, idx32, swr)
    return jnp.transpose(_log_kernel(st))
